# baseline (speedup 1.0000x reference)
import jax
import jax.numpy as jnp
from jax import lax
from jax.experimental import pallas as pl
from jax.experimental.pallas import tpu as pltpu
from jax.experimental.pallas import tpu_sc as plsc

_B = 16384
_D = 32
_V = 1000000
_NC = 2
_NS = 16
_NW = _NC * _NS
_SLABW = 512
_SHIFT = 9
_NSLAB = (_V + _SLABW - 1) // _SLABW
_SPT = (_NSLAB + _NW - 1) // _NW
_WIN = 512
_TAILW = 128
_LCAP = 768
_CAP = 32


def _sc_body(uidx_hbm, iidx_hbm, tum_hbm, tim_hbm, tuf_hbm, tif_hbm, s2a_hbm,
             tum_tail, tim_tail, tuf_tail, tif_tail,
             out_u, out_c, out_i,
             uwin, iwin, cwin,
             uvals, upos, cvals, cpos, ivals, ipos,
             ubinc, ubinp, cbinc, cbinp, ibinc, ibinp, ucnt, ccnt, icnt,
             sa0, sa1, sb0, sb1, rowbuf, rowbuf2, posb, posb2,
             semp0, semp1, sem2):
    t = lax.axis_index("s") * _NC + lax.axis_index("c")
    iota16 = lax.iota(jnp.int32, 16)
    neg1 = jnp.full((16,), -1, jnp.int32)

    def compact_window(base_b, win_ref, vals_ref, pos_ref, off):
        def chunk(k, off):
            v = win_ref[pl.ds(k * 16, 16)]
            s = lax.shift_right_logical(v, _SHIFT)
            m = (s & (_NW - 1)) == t
            cnt = plsc.all_reduce_population_count(m)
            wp = off + plsc.cumsum(m.astype(jnp.int32)) - 1
            plsc.store_scatter(vals_ref, [wp], v, mask=m)
            plsc.store_scatter(pos_ref, [wp], base_b + k * 16 + iota16, mask=m)
            return off + cnt[0]
        return lax.fori_loop(0, _WIN // 16, chunk, off)

    def awin(w, offs):
        uo, co, io = offs
        b0 = w * _WIN
        pltpu.sync_copy(uidx_hbm.at[pl.ds(b0, _WIN)], uwin)
        pltpu.sync_copy(iidx_hbm.at[pl.ds(b0, _WIN)], iwin)
        cps = [pltpu.async_copy(s2a_hbm.at[uwin.at[pl.ds(j * 128, 128)]],
                                cwin.at[pl.ds(j * 128, 128)], sem2)
               for j in range(_WIN // 128)]
        for cp in cps:
            cp.wait()
        uo = compact_window(b0, uwin, uvals, upos, uo)
        co = compact_window(b0, cwin, cvals, cpos, co)
        io = compact_window(b0, iwin, ivals, ipos, io)
        return (uo, co, io)

    nu, ncr, ni = lax.fori_loop(0, _B // _WIN, awin,
                                (jnp.int32(0), jnp.int32(0), jnp.int32(0)))

    zero16 = jnp.zeros((16,), jnp.int32)
    for cref in (ucnt, ccnt, icnt):
        for q in range(4):
            cref[pl.ds(q * 16, 16)] = zero16

    def bin_list(vals_ref, pos_ref, n, binc, binp, cnt_ref):
        def chunk(k, _):
            valid = k * 16 + iota16 < n
            v = vals_ref[pl.ds(k * 16, 16)]
            bp = pos_ref[pl.ds(k * 16, 16)]
            o = lax.shift_right_logical(v, 14)
            rank, lastm = plsc.scan_count(o, valid)
            base = plsc.load_gather(cnt_ref, [o], mask=valid)
            slot = jnp.minimum(base + rank - 1, _CAP - 1)
            plsc.store_scatter(binc, [o, slot], v & (_SLABW - 1), mask=valid)
            plsc.store_scatter(binp, [o, slot], bp, mask=valid)
            plsc.store_scatter(cnt_ref, [o], slot + 1, mask=lastm & valid)
            return 0

        lax.fori_loop(0, (n + 15) // 16, chunk, 0)

    bin_list(uvals, upos, nu, ubinc, ubinp, ucnt)
    bin_list(cvals, cpos, ncr, cbinc, cbinp, ccnt)
    bin_list(ivals, ipos, ni, ibinc, ibinp, icnt)

    def fire(tbl_a, tbl_b, tail_a, tail_b, sa, sb, semp, si):
        s = si * _NW + t

        @pl.when(s < _NSLAB - 1)
        def _():
            base = s * _SLABW
            for h in range(_SLABW // 128):
                pltpu.async_copy(tbl_a.at[:, pl.ds(base + h * 128, 128)],
                                 sa.at[:, pl.ds(h * 128, 128)], semp)
                pltpu.async_copy(tbl_b.at[:, pl.ds(base + h * 128, 128)],
                                 sb.at[:, pl.ds(h * 128, 128)], semp)

        @pl.when(s == _NSLAB - 1)
        def _():
            pltpu.async_copy(tail_a, sa.at[:, pl.ds(0, _TAILW)], semp)
            pltpu.async_copy(tail_b, sb.at[:, pl.ds(0, _TAILW)], semp)

    def drain(tbl_a, tail_a, sa, sb, semp, si):
        s = si * _NW + t

        @pl.when(s < _NSLAB - 1)
        def _():
            pltpu.make_async_copy(tbl_a.at[:, pl.ds(0, _SLABW)], sa, semp).wait()
            pltpu.make_async_copy(tbl_a.at[:, pl.ds(0, _SLABW)], sb, semp).wait()

        @pl.when(s == _NSLAB - 1)
        def _():
            pltpu.make_async_copy(tail_a, sa.at[:, pl.ds(0, _TAILW)], semp).wait()
            pltpu.make_async_copy(tail_a, sb.at[:, pl.ds(0, _TAILW)], semp).wait()

    def extract_set(si, sa, sb, binc, binp, cnt_ref, rb, pb, out_hbm):
        for q in range(_CAP // 16):
            pb[pl.ds(q * 16, 16)] = neg1
        cc = cnt_ref[pl.ds((si // 16) * 16, 16)]
        nsub = jnp.minimum(
            lax.reduce_max(jnp.where(iota16 == si % 16, cc, 0), (0,)), _CAP)

        def group(g, _):
            cols = binc[si, pl.ds(g * 16, 16)]
            bp = binp[si, pl.ds(g * 16, 16)]
            lanes = g * 16 + iota16
            gm = lanes < nsub
            plsc.store_scatter(pb, [lanes], bp, mask=gm)
            for d in range(_D):
                dd = jnp.broadcast_to(jnp.int32(d), (16,))
                va = plsc.load_gather(sa, [dd, cols], mask=gm)
                plsc.store_scatter(rb, [lanes, dd], va, mask=gm)
                vb = plsc.load_gather(sb, [dd, cols], mask=gm)
                plsc.store_scatter(rb, [lanes, dd + _D], vb, mask=gm)
            return 0

        lax.fori_loop(0, (nsub + 15) // 16, group, 0)
        return pltpu.async_copy(
            rb, out_hbm.at[plsc.Indices(pb, ignored_value=-1)], sem2)

    def run_pass(tbl_a, tbl_b, tail_a, tail_b, sets):
        @pl.when(t < _NSLAB)
        def _():
            fire(tbl_a, tbl_b, tail_a, tail_b, sa0, sb0, semp0, 0)

        def unit(si, sa, sb, semp, nsa, nsb, nsemp):
            s = si * _NW + t

            @pl.when((si + 1) * _NW + t < _NSLAB)
            def _():
                fire(tbl_a, tbl_b, tail_a, tail_b, nsa, nsb, nsemp, si + 1)

            @pl.when(s < _NSLAB)
            def _():
                drain(tbl_a, tail_a, sa, sb, semp, si)
                cps = [extract_set(si, sa, sb, *st) for st in sets]
                for cp in cps:
                    cp.wait()

        def outer(j, _):
            unit(2 * j, sa0, sb0, semp0, sa1, sb1, semp1)
            unit(2 * j + 1, sa1, sb1, semp1, sa0, sb0, semp0)
            return 0

        lax.fori_loop(0, _SPT // 2, outer, 0)

    run_pass(tum_hbm, tuf_hbm, tum_tail, tuf_tail,
             [(ubinc, ubinp, ucnt, rowbuf, posb, out_u),
              (cbinc, cbinp, ccnt, rowbuf2, posb2, out_c)])
    run_pass(tim_hbm, tif_hbm, tim_tail, tif_tail,
             [(ibinc, ibinp, icnt, rowbuf, posb, out_i)])


_sc_gather = pl.kernel(
    _sc_body,
    out_type=[
        jax.ShapeDtypeStruct((_B, 128), jnp.float32),
        jax.ShapeDtypeStruct((_B, 128), jnp.float32),
        jax.ShapeDtypeStruct((_B, 128), jnp.float32),
    ],
    mesh=plsc.VectorSubcoreMesh(core_axis_name="c", subcore_axis_name="s",
                                num_cores=_NC, num_subcores=_NS),
    scratch_types=[
        pltpu.VMEM((_WIN,), jnp.int32),
        pltpu.VMEM((_WIN,), jnp.int32),
        pltpu.VMEM((_WIN,), jnp.int32),
        pltpu.VMEM((_LCAP,), jnp.int32),
        pltpu.VMEM((_LCAP,), jnp.int32),
        pltpu.VMEM((_LCAP,), jnp.int32),
        pltpu.VMEM((_LCAP,), jnp.int32),
        pltpu.VMEM((_LCAP,), jnp.int32),
        pltpu.VMEM((_LCAP,), jnp.int32),
        pltpu.VMEM((_SPT, _CAP), jnp.int32),
        pltpu.VMEM((_SPT, _CAP), jnp.int32),
        pltpu.VMEM((_SPT, _CAP), jnp.int32),
        pltpu.VMEM((_SPT, _CAP), jnp.int32),
        pltpu.VMEM((_SPT, _CAP), jnp.int32),
        pltpu.VMEM((_SPT, _CAP), jnp.int32),
        pltpu.VMEM((64,), jnp.int32),
        pltpu.VMEM((64,), jnp.int32),
        pltpu.VMEM((64,), jnp.int32),
        pltpu.VMEM((_D, _SLABW), jnp.float32),
        pltpu.VMEM((_D, _SLABW), jnp.float32),
        pltpu.VMEM((_D, _SLABW), jnp.float32),
        pltpu.VMEM((_D, _SLABW), jnp.float32),
        pltpu.VMEM((_CAP, 128), jnp.float32),
        pltpu.VMEM((_CAP, 128), jnp.float32),
        pltpu.VMEM((_CAP,), jnp.int32),
        pltpu.VMEM((_CAP,), jnp.int32),
        pltpu.SemaphoreType.DMA,
        pltpu.SemaphoreType.DMA,
        pltpu.SemaphoreType.DMA,
    ],
    compiler_params=pltpu.CompilerParams(needs_layout_passes=False),
)

_R = 2048
_NSTEP = _B // _R


def _tc_body(gu, gc, gi, lab, w1u, w1i, b1, w2, b2, w3, b3, wamT, wamfT,
             ba, rating_out, scal_out, acc_ref):
    i = pl.program_id(0)

    @pl.when(i == 0)
    def _():
        acc_ref[0] = 0.0
        acc_ref[1] = 0.0

    f32 = jnp.float32
    u = gu[...]
    c = gc[...]
    um = u[:, :_D]
    uf = u[:, _D:2 * _D]
    im = gi[:, :_D]
    imf = gi[:, _D:2 * _D]
    mf = uf * imf
    h1 = jnp.maximum(
        jnp.dot(um, w1u[...], preferred_element_type=f32)
        + jnp.dot(im, w1i[...], preferred_element_type=f32) + b1[...], 0.0)
    h2 = jnp.maximum(jnp.dot(h1, w2[...], preferred_element_type=f32) + b2[...], 0.0)
    h3 = jnp.maximum(jnp.dot(h2, w3[...], preferred_element_type=f32) + b3[...], 0.0)
    dn = (((1,), (1,)), ((), ()))
    lrow = (lax.dot_general(wamT[...], h3, dn, preferred_element_type=f32)
            + lax.dot_general(wamfT[...], mf, dn, preferred_element_type=f32)
            + ba[0])
    y = lab[...]
    rating_out[...] = 1.0 / (1.0 + jnp.exp(-lrow))
    bce = jnp.maximum(lrow, 0.0) - lrow * y + jnp.log1p(jnp.exp(-jnp.abs(lrow)))
    d = u[:, :2 * _D] - c[:, :2 * _D]
    acc_ref[0] += jnp.sum(bce)
    acc_ref[1] += jnp.sum(d * d)

    @pl.when(i == _NSTEP - 1)
    def _():
        obce = acc_ref[0] / _B
        ot = acc_ref[1] / _B
        scal_out[0] = obce + ot
        scal_out[1] = obce
        scal_out[2] = ot


_tc_dense = pl.pallas_call(
    _tc_body,
    grid=(_NSTEP,),
    in_specs=[
        pl.BlockSpec((_R, 128), lambda i: (i, 0)),
        pl.BlockSpec((_R, 128), lambda i: (i, 0)),
        pl.BlockSpec((_R, 128), lambda i: (i, 0)),
        pl.BlockSpec((1, _R), lambda i: (0, i)),
        pl.BlockSpec((_D, _D), lambda i: (0, 0)),
        pl.BlockSpec((_D, _D), lambda i: (0, 0)),
        pl.BlockSpec((1, _D), lambda i: (0, 0)),
        pl.BlockSpec((_D, 16), lambda i: (0, 0)),
        pl.BlockSpec((1, 16), lambda i: (0, 0)),
        pl.BlockSpec((16, 8), lambda i: (0, 0)),
        pl.BlockSpec((1, 8), lambda i: (0, 0)),
        pl.BlockSpec((1, 8), lambda i: (0, 0)),
        pl.BlockSpec((1, _D), lambda i: (0, 0)),
        pl.BlockSpec(memory_space=pltpu.SMEM),
    ],
    out_specs=[
        pl.BlockSpec((1, _R), lambda i: (0, i)),
        pl.BlockSpec(memory_space=pltpu.SMEM),
    ],
    out_shape=[
        jax.ShapeDtypeStruct((1, _B), jnp.float32),
        jax.ShapeDtypeStruct((3,), jnp.float32),
    ],
    scratch_shapes=[pltpu.SMEM((2,), jnp.float32)],
)


def _tail(tT):
    w = _V - (_NSLAB - 1) * _SLABW
    return jnp.pad(tT[:, (_NSLAB - 1) * _SLABW:], ((0, 0), (0, _TAILW - w)))


def kernel(user_indices, item_indices, labels, emb_user_mlp, emb_item_mlp,
           emb_user_mf, emb_item_mf, W1, b1, W2, b2, W3, b3, Wa, ba, s2a_map):
    tum, tim, tuf, tif = (emb_user_mlp.T, emb_item_mlp.T,
                          emb_user_mf.T, emb_item_mf.T)
    gu, gc, gi = _sc_gather(
        user_indices, item_indices, tum, tim, tuf, tif, s2a_map,
        _tail(tum), _tail(tim), _tail(tuf), _tail(tif))
    rating_row, scal = _tc_dense(
        gu, gc, gi, labels.reshape(1, _B),
        W1[:_D], W1[_D:], b1.reshape(1, _D),
        W2, b2.reshape(1, 16), W3, b3.reshape(1, 8),
        Wa[:8].T, Wa[8:].T, ba)
    rating = rating_row.reshape(_B)
    return (scal[0], scal[1], scal[2], rating, labels)


# speedup vs baseline: 1.1166x; 1.1166x over previous
import jax
import jax.numpy as jnp
from jax import lax
from jax.experimental import pallas as pl
from jax.experimental.pallas import tpu as pltpu
from jax.experimental.pallas import tpu_sc as plsc

_B = 16384
_D = 32
_V = 1000000
_NC = 2
_NS = 16
_NW = _NC * _NS
_SLABW = 512
_SHIFT = 9
_NSLAB = (_V + _SLABW - 1) // _SLABW
_SPT = (_NSLAB + _NW - 1) // _NW
_WIN = 1024
_TAILW = 128
_LCAP = 640
_CAP = 32


def _sc_body(uidx_hbm, iidx_hbm, tum_hbm, tim_hbm, tuf_hbm, tif_hbm, s2a_hbm,
             tum_tail, tim_tail, tuf_tail, tif_tail,
             out_u, out_c, out_i,
             uwin, iwin,
             uvals, upos, cvals, cpos, ivals, ipos,
             ubinc, ubinp, cbinc, cbinp, ibinc, ibinp, ucnt, ccnt, icnt,
             sa0, sa1, sb0, sb1, rowbuf, rowbuf2, posb, posb2,
             semp0, semp1, sem2):
    t = lax.axis_index("s") * _NC + lax.axis_index("c")
    iota16 = lax.iota(jnp.int32, 16)
    neg1 = jnp.full((16,), -1, jnp.int32)

    def compact_window(base_b, win_ref, vals_ref, pos_ref, off):
        def chunk(k, off):
            v = win_ref[pl.ds(k * 16, 16)]
            s = lax.shift_right_logical(v, _SHIFT)
            m = (s & (_NW - 1)) == t
            cnt = plsc.all_reduce_population_count(m)
            wp = off + plsc.cumsum(m.astype(jnp.int32)) - 1
            plsc.store_scatter(vals_ref, [wp], v, mask=m)
            plsc.store_scatter(pos_ref, [wp], base_b + k * 16 + iota16, mask=m)
            return off + cnt[0]
        return lax.fori_loop(0, _WIN // 16, chunk, off, unroll=2)

    def awin(w, offs):
        uo, co, io = offs
        b0 = w * _WIN
        pltpu.sync_copy(uidx_hbm.at[pl.ds(b0, _WIN)], uwin)
        pltpu.sync_copy(iidx_hbm.at[pl.ds(b0, _WIN)], iwin)
        io = compact_window(b0, iwin, ivals, ipos, io)
        cps = [pltpu.async_copy(s2a_hbm.at[uwin.at[pl.ds(j * 128, 128)]],
                                iwin.at[pl.ds(j * 128, 128)], sem2)
               for j in range(_WIN // 128)]
        uo = compact_window(b0, uwin, uvals, upos, uo)
        for cp in cps:
            cp.wait()
        co = compact_window(b0, iwin, cvals, cpos, co)
        return (uo, co, io)

    nu, ncr, ni = lax.fori_loop(0, _B // _WIN, awin,
                                (jnp.int32(0), jnp.int32(0), jnp.int32(0)))

    zero16 = jnp.zeros((16,), jnp.int32)
    for cref in (ucnt, ccnt, icnt):
        for q in range(4):
            cref[pl.ds(q * 16, 16)] = zero16

    def bin_list(vals_ref, pos_ref, n, binc, binp, cnt_ref):
        def chunk(k, _):
            valid = k * 16 + iota16 < n
            v = vals_ref[pl.ds(k * 16, 16)]
            bp = pos_ref[pl.ds(k * 16, 16)]
            o = lax.shift_right_logical(v, 14)
            rank, lastm = plsc.scan_count(o, valid)
            base = plsc.load_gather(cnt_ref, [o], mask=valid)
            slot = jnp.minimum(base + rank - 1, _CAP - 1)
            plsc.store_scatter(binc, [o, slot], v & (_SLABW - 1), mask=valid)
            plsc.store_scatter(binp, [o, slot], bp, mask=valid)
            plsc.store_scatter(cnt_ref, [o], slot + 1, mask=lastm & valid)
            return 0

        lax.fori_loop(0, (n + 15) // 16, chunk, 0)

    bin_list(uvals, upos, nu, ubinc, ubinp, ucnt)
    bin_list(cvals, cpos, ncr, cbinc, cbinp, ccnt)
    bin_list(ivals, ipos, ni, ibinc, ibinp, icnt)

    def fire(tbl_a, tbl_b, tail_a, tail_b, sa, sb, semp, si):
        s = si * _NW + t

        @pl.when(s < _NSLAB - 1)
        def _():
            base = s * _SLABW
            for h in range(_SLABW // 128):
                pltpu.async_copy(tbl_a.at[:, pl.ds(base + h * 128, 128)],
                                 sa.at[:, pl.ds(h * 128, 128)], semp)
                pltpu.async_copy(tbl_b.at[:, pl.ds(base + h * 128, 128)],
                                 sb.at[:, pl.ds(h * 128, 128)], semp)

        @pl.when(s == _NSLAB - 1)
        def _():
            pltpu.async_copy(tail_a, sa.at[:, pl.ds(0, _TAILW)], semp)
            pltpu.async_copy(tail_b, sb.at[:, pl.ds(0, _TAILW)], semp)

    def drain(tbl_a, tail_a, sa, sb, semp, si):
        s = si * _NW + t

        @pl.when(s < _NSLAB - 1)
        def _():
            pltpu.make_async_copy(tbl_a.at[:, pl.ds(0, _SLABW)], sa, semp).wait()
            pltpu.make_async_copy(tbl_a.at[:, pl.ds(0, _SLABW)], sb, semp).wait()

        @pl.when(s == _NSLAB - 1)
        def _():
            pltpu.make_async_copy(tail_a, sa.at[:, pl.ds(0, _TAILW)], semp).wait()
            pltpu.make_async_copy(tail_a, sb.at[:, pl.ds(0, _TAILW)], semp).wait()

    def extract_set(si, sa, sb, binc, binp, cnt_ref, rb, pb, out_hbm):
        for q in range(_CAP // 16):
            pb[pl.ds(q * 16, 16)] = neg1
        cc = cnt_ref[pl.ds((si // 16) * 16, 16)]
        nsub = jnp.minimum(
            lax.reduce_max(jnp.where(iota16 == si % 16, cc, 0), (0,)), _CAP)

        def group(g, _):
            cols = binc[si, pl.ds(g * 16, 16)]
            bp = binp[si, pl.ds(g * 16, 16)]
            lanes = g * 16 + iota16
            gm = lanes < nsub
            plsc.store_scatter(pb, [lanes], bp, mask=gm)
            for d in range(_D):
                dd = jnp.broadcast_to(jnp.int32(d), (16,))
                va = plsc.load_gather(sa, [dd, cols], mask=gm)
                plsc.store_scatter(rb, [lanes, dd], va, mask=gm)
                vb = plsc.load_gather(sb, [dd, cols], mask=gm)
                plsc.store_scatter(rb, [lanes, dd + _D], vb, mask=gm)
            return 0

        lax.fori_loop(0, (nsub + 15) // 16, group, 0)
        return pltpu.async_copy(
            rb, out_hbm.at[plsc.Indices(pb, ignored_value=-1)], sem2)

    def run_pass(tbl_a, tbl_b, tail_a, tail_b, sets):
        @pl.when(t < _NSLAB)
        def _():
            fire(tbl_a, tbl_b, tail_a, tail_b, sa0, sb0, semp0, 0)

        def unit(si, sa, sb, semp, nsa, nsb, nsemp):
            s = si * _NW + t

            @pl.when((si + 1) * _NW + t < _NSLAB)
            def _():
                fire(tbl_a, tbl_b, tail_a, tail_b, nsa, nsb, nsemp, si + 1)

            @pl.when(s < _NSLAB)
            def _():
                drain(tbl_a, tail_a, sa, sb, semp, si)
                cps = [extract_set(si, sa, sb, *st) for st in sets]
                for cp in cps:
                    cp.wait()

        def outer(j, _):
            unit(2 * j, sa0, sb0, semp0, sa1, sb1, semp1)
            unit(2 * j + 1, sa1, sb1, semp1, sa0, sb0, semp0)
            return 0

        lax.fori_loop(0, _SPT // 2, outer, 0)

    run_pass(tum_hbm, tuf_hbm, tum_tail, tuf_tail,
             [(ubinc, ubinp, ucnt, rowbuf, posb, out_u),
              (cbinc, cbinp, ccnt, rowbuf2, posb2, out_c)])
    run_pass(tim_hbm, tif_hbm, tim_tail, tif_tail,
             [(ibinc, ibinp, icnt, rowbuf, posb, out_i)])


_sc_gather = pl.kernel(
    _sc_body,
    out_type=[
        jax.ShapeDtypeStruct((_B, 128), jnp.float32),
        jax.ShapeDtypeStruct((_B, 128), jnp.float32),
        jax.ShapeDtypeStruct((_B, 128), jnp.float32),
    ],
    mesh=plsc.VectorSubcoreMesh(core_axis_name="c", subcore_axis_name="s",
                                num_cores=_NC, num_subcores=_NS),
    scratch_types=[
        pltpu.VMEM((_WIN,), jnp.int32),
        pltpu.VMEM((_WIN,), jnp.int32),
        pltpu.VMEM((_LCAP,), jnp.int32),
        pltpu.VMEM((_LCAP,), jnp.int32),
        pltpu.VMEM((_LCAP,), jnp.int32),
        pltpu.VMEM((_LCAP,), jnp.int32),
        pltpu.VMEM((_LCAP,), jnp.int32),
        pltpu.VMEM((_LCAP,), jnp.int32),
        pltpu.VMEM((_SPT, _CAP), jnp.int32),
        pltpu.VMEM((_SPT, _CAP), jnp.int32),
        pltpu.VMEM((_SPT, _CAP), jnp.int32),
        pltpu.VMEM((_SPT, _CAP), jnp.int32),
        pltpu.VMEM((_SPT, _CAP), jnp.int32),
        pltpu.VMEM((_SPT, _CAP), jnp.int32),
        pltpu.VMEM((64,), jnp.int32),
        pltpu.VMEM((64,), jnp.int32),
        pltpu.VMEM((64,), jnp.int32),
        pltpu.VMEM((_D, _SLABW), jnp.float32),
        pltpu.VMEM((_D, _SLABW), jnp.float32),
        pltpu.VMEM((_D, _SLABW), jnp.float32),
        pltpu.VMEM((_D, _SLABW), jnp.float32),
        pltpu.VMEM((_CAP, 128), jnp.float32),
        pltpu.VMEM((_CAP, 128), jnp.float32),
        pltpu.VMEM((_CAP,), jnp.int32),
        pltpu.VMEM((_CAP,), jnp.int32),
        pltpu.SemaphoreType.DMA,
        pltpu.SemaphoreType.DMA,
        pltpu.SemaphoreType.DMA,
    ],
    compiler_params=pltpu.CompilerParams(needs_layout_passes=False),
)

_R = 2048
_NSTEP = _B // _R


def _tc_body(gu, gc, gi, lab, w1u, w1i, b1, w2, b2, w3, b3, wamT, wamfT,
             ba, rating_out, scal_out, acc_ref):
    i = pl.program_id(0)

    @pl.when(i == 0)
    def _():
        acc_ref[0] = 0.0
        acc_ref[1] = 0.0

    f32 = jnp.float32
    u = gu[...]
    c = gc[...]
    um = u[:, :_D]
    uf = u[:, _D:2 * _D]
    im = gi[:, :_D]
    imf = gi[:, _D:2 * _D]
    mf = uf * imf
    h1 = jnp.maximum(
        jnp.dot(um, w1u[...], preferred_element_type=f32)
        + jnp.dot(im, w1i[...], preferred_element_type=f32) + b1[...], 0.0)
    h2 = jnp.maximum(jnp.dot(h1, w2[...], preferred_element_type=f32) + b2[...], 0.0)
    h3 = jnp.maximum(jnp.dot(h2, w3[...], preferred_element_type=f32) + b3[...], 0.0)
    dn = (((1,), (1,)), ((), ()))
    lrow = (lax.dot_general(wamT[...], h3, dn, preferred_element_type=f32)
            + lax.dot_general(wamfT[...], mf, dn, preferred_element_type=f32)
            + ba[0])
    y = lab[...]
    rating_out[...] = 1.0 / (1.0 + jnp.exp(-lrow))
    bce = jnp.maximum(lrow, 0.0) - lrow * y + jnp.log1p(jnp.exp(-jnp.abs(lrow)))
    d = u[:, :2 * _D] - c[:, :2 * _D]
    acc_ref[0] += jnp.sum(bce)
    acc_ref[1] += jnp.sum(d * d)

    @pl.when(i == _NSTEP - 1)
    def _():
        obce = acc_ref[0] / _B
        ot = acc_ref[1] / _B
        scal_out[0] = obce + ot
        scal_out[1] = obce
        scal_out[2] = ot


_tc_dense = pl.pallas_call(
    _tc_body,
    grid=(_NSTEP,),
    in_specs=[
        pl.BlockSpec((_R, 128), lambda i: (i, 0)),
        pl.BlockSpec((_R, 128), lambda i: (i, 0)),
        pl.BlockSpec((_R, 128), lambda i: (i, 0)),
        pl.BlockSpec((1, _R), lambda i: (0, i)),
        pl.BlockSpec((_D, _D), lambda i: (0, 0)),
        pl.BlockSpec((_D, _D), lambda i: (0, 0)),
        pl.BlockSpec((1, _D), lambda i: (0, 0)),
        pl.BlockSpec((_D, 16), lambda i: (0, 0)),
        pl.BlockSpec((1, 16), lambda i: (0, 0)),
        pl.BlockSpec((16, 8), lambda i: (0, 0)),
        pl.BlockSpec((1, 8), lambda i: (0, 0)),
        pl.BlockSpec((1, 8), lambda i: (0, 0)),
        pl.BlockSpec((1, _D), lambda i: (0, 0)),
        pl.BlockSpec(memory_space=pltpu.SMEM),
    ],
    out_specs=[
        pl.BlockSpec((1, _R), lambda i: (0, i)),
        pl.BlockSpec(memory_space=pltpu.SMEM),
    ],
    out_shape=[
        jax.ShapeDtypeStruct((1, _B), jnp.float32),
        jax.ShapeDtypeStruct((3,), jnp.float32),
    ],
    scratch_shapes=[pltpu.SMEM((2,), jnp.float32)],
)


def _tail(tT):
    w = _V - (_NSLAB - 1) * _SLABW
    return jnp.pad(tT[:, (_NSLAB - 1) * _SLABW:], ((0, 0), (0, _TAILW - w)))


def kernel(user_indices, item_indices, labels, emb_user_mlp, emb_item_mlp,
           emb_user_mf, emb_item_mf, W1, b1, W2, b2, W3, b3, Wa, ba, s2a_map):
    tum, tim, tuf, tif = (emb_user_mlp.T, emb_item_mlp.T,
                          emb_user_mf.T, emb_item_mf.T)
    gu, gc, gi = _sc_gather(
        user_indices, item_indices, tum, tim, tuf, tif, s2a_map,
        _tail(tum), _tail(tim), _tail(tuf), _tail(tif))
    rating_row, scal = _tc_dense(
        gu, gc, gi, labels.reshape(1, _B),
        W1[:_D], W1[_D:], b1.reshape(1, _D),
        W2, b2.reshape(1, 16), W3, b3.reshape(1, 8),
        Wa[:8].T, Wa[8:].T, ba)
    rating = rating_row.reshape(_B)
    return (scal[0], scal[1], scal[2], rating, labels)


# speedup vs baseline: 1.1209x; 1.0038x over previous
import jax
import jax.numpy as jnp
from jax import lax
from jax.experimental import pallas as pl
from jax.experimental.pallas import tpu as pltpu
from jax.experimental.pallas import tpu_sc as plsc

_B = 16384
_D = 32
_V = 1000000
_NC = 2
_NS = 16
_NW = _NC * _NS
_SLABW = 512
_SHIFT = 9
_NSLAB = (_V + _SLABW - 1) // _SLABW
_SPT = (_NSLAB + _NW - 1) // _NW
_WIN = 1024
_TAILW = 128
_LCAP = 640
_CAP = 32


def _sc_body(uidx_hbm, iidx_hbm, tum_hbm, tim_hbm, tuf_hbm, tif_hbm, s2a_hbm,
             tum_tail, tim_tail, tuf_tail, tif_tail,
             out_u, out_c, out_i,
             uwin, iwin,
             uvals, upos, cvals, cpos, ivals, ipos,
             ubinc, ubinp, cbinc, cbinp, ibinc, ibinp, ucnt, ccnt, icnt,
             sa0, sa1, sb0, sb1, rowbuf, rowbuf2, posb, posb2,
             semp0, semp1, sem2):
    t = lax.axis_index("s") * _NC + lax.axis_index("c")
    iota16 = lax.iota(jnp.int32, 16)
    neg1 = jnp.full((16,), -1, jnp.int32)

    def fire(tbl_a, tbl_b, tail_a, tail_b, sa, sb, semp, si):
        s = si * _NW + t

        @pl.when(s < _NSLAB - 1)
        def _():
            base = s * _SLABW
            for h in range(_SLABW // 128):
                pltpu.async_copy(tbl_a.at[:, pl.ds(base + h * 128, 128)],
                                 sa.at[:, pl.ds(h * 128, 128)], semp)
                pltpu.async_copy(tbl_b.at[:, pl.ds(base + h * 128, 128)],
                                 sb.at[:, pl.ds(h * 128, 128)], semp)

        @pl.when(s == _NSLAB - 1)
        def _():
            pltpu.async_copy(tail_a, sa.at[:, pl.ds(0, _TAILW)], semp)
            pltpu.async_copy(tail_b, sb.at[:, pl.ds(0, _TAILW)], semp)

    def drain(tbl_a, tail_a, sa, sb, semp, si):
        s = si * _NW + t

        @pl.when(s < _NSLAB - 1)
        def _():
            pltpu.make_async_copy(tbl_a.at[:, pl.ds(0, _SLABW)], sa, semp).wait()
            pltpu.make_async_copy(tbl_a.at[:, pl.ds(0, _SLABW)], sb, semp).wait()

        @pl.when(s == _NSLAB - 1)
        def _():
            pltpu.make_async_copy(tail_a, sa.at[:, pl.ds(0, _TAILW)], semp).wait()
            pltpu.make_async_copy(tail_a, sb.at[:, pl.ds(0, _TAILW)], semp).wait()


    fire(tum_hbm, tuf_hbm, tum_tail, tuf_tail, sa0, sb0, semp0, 0)
    fire(tum_hbm, tuf_hbm, tum_tail, tuf_tail, sa1, sb1, semp1, 1)

    def compact_window(base_b, win_ref, vals_ref, pos_ref, off):
        def chunk(k, off):
            v = win_ref[pl.ds(k * 16, 16)]
            s = lax.shift_right_logical(v, _SHIFT)
            m = (s & (_NW - 1)) == t
            cnt = plsc.all_reduce_population_count(m)
            wp = off + plsc.cumsum(m.astype(jnp.int32)) - 1
            plsc.store_scatter(vals_ref, [wp], v, mask=m)
            plsc.store_scatter(pos_ref, [wp], base_b + k * 16 + iota16, mask=m)
            return off + cnt[0]
        return lax.fori_loop(0, _WIN // 16, chunk, off, unroll=2)

    def awin(w, offs):
        uo, co, io = offs
        b0 = w * _WIN
        pltpu.sync_copy(uidx_hbm.at[pl.ds(b0, _WIN)], uwin)
        pltpu.sync_copy(iidx_hbm.at[pl.ds(b0, _WIN)], iwin)
        io = compact_window(b0, iwin, ivals, ipos, io)
        cps = [pltpu.async_copy(s2a_hbm.at[uwin.at[pl.ds(j * 128, 128)]],
                                iwin.at[pl.ds(j * 128, 128)], sem2)
               for j in range(_WIN // 128)]
        uo = compact_window(b0, uwin, uvals, upos, uo)
        for cp in cps:
            cp.wait()
        co = compact_window(b0, iwin, cvals, cpos, co)
        return (uo, co, io)

    nu, ncr, ni = lax.fori_loop(0, _B // _WIN, awin,
                                (jnp.int32(0), jnp.int32(0), jnp.int32(0)))

    zero16 = jnp.zeros((16,), jnp.int32)
    for cref in (ucnt, ccnt, icnt):
        for q in range(4):
            cref[pl.ds(q * 16, 16)] = zero16

    def bin_list(vals_ref, pos_ref, n, binc, binp, cnt_ref):
        def chunk(k, _):
            valid = k * 16 + iota16 < n
            v = vals_ref[pl.ds(k * 16, 16)]
            bp = pos_ref[pl.ds(k * 16, 16)]
            o = lax.shift_right_logical(v, 14)
            rank, lastm = plsc.scan_count(o, valid)
            base = plsc.load_gather(cnt_ref, [o], mask=valid)
            slot = jnp.minimum(base + rank - 1, _CAP - 1)
            plsc.store_scatter(binc, [o, slot], v & (_SLABW - 1), mask=valid)
            plsc.store_scatter(binp, [o, slot], bp, mask=valid)
            plsc.store_scatter(cnt_ref, [o], slot + 1, mask=lastm & valid)
            return 0

        lax.fori_loop(0, (n + 15) // 16, chunk, 0)

    bin_list(uvals, upos, nu, ubinc, ubinp, ucnt)
    bin_list(cvals, cpos, ncr, cbinc, cbinp, ccnt)
    bin_list(ivals, ipos, ni, ibinc, ibinp, icnt)

    def extract_set(si, sa, sb, binc, binp, cnt_ref, rb, pb, out_hbm):
        for q in range(_CAP // 16):
            pb[pl.ds(q * 16, 16)] = neg1
        cc = cnt_ref[pl.ds((si // 16) * 16, 16)]
        nsub = jnp.minimum(
            lax.reduce_max(jnp.where(iota16 == si % 16, cc, 0), (0,)), _CAP)

        def group(g, _):
            cols = binc[si, pl.ds(g * 16, 16)]
            bp = binp[si, pl.ds(g * 16, 16)]
            lanes = g * 16 + iota16
            gm = lanes < nsub
            plsc.store_scatter(pb, [lanes], bp, mask=gm)
            for d in range(_D):
                dd = jnp.broadcast_to(jnp.int32(d), (16,))
                va = plsc.load_gather(sa, [dd, cols], mask=gm)
                plsc.store_scatter(rb, [lanes, dd], va, mask=gm)
                vb = plsc.load_gather(sb, [dd, cols], mask=gm)
                plsc.store_scatter(rb, [lanes, dd + _D], vb, mask=gm)
            return 0

        lax.fori_loop(0, (nsub + 15) // 16, group, 0)
        return pltpu.async_copy(
            rb, out_hbm.at[plsc.Indices(pb, ignored_value=-1)], sem2)

    def run_pass(tbl_a, tbl_b, tail_a, tail_b, sets, primed_hi=-1):
        if primed_hi < 0:
            fire(tbl_a, tbl_b, tail_a, tail_b, sa0, sb0, semp0, 0)

        def unit(si, sa, sb, semp, nsa, nsb, nsemp):
            s = si * _NW + t

            @pl.when(((si + 1) * _NW + t < _NSLAB) & (si + 1 > primed_hi))
            def _():
                fire(tbl_a, tbl_b, tail_a, tail_b, nsa, nsb, nsemp, si + 1)

            @pl.when(s < _NSLAB)
            def _():
                drain(tbl_a, tail_a, sa, sb, semp, si)
                cps = [extract_set(si, sa, sb, *st) for st in sets]
                for cp in cps:
                    cp.wait()

        def outer(j, _):
            unit(2 * j, sa0, sb0, semp0, sa1, sb1, semp1)
            unit(2 * j + 1, sa1, sb1, semp1, sa0, sb0, semp0)
            return 0

        lax.fori_loop(0, _SPT // 2, outer, 0)

    run_pass(tum_hbm, tuf_hbm, tum_tail, tuf_tail,
             [(ubinc, ubinp, ucnt, rowbuf, posb, out_u),
              (cbinc, cbinp, ccnt, rowbuf2, posb2, out_c)], primed_hi=1)
    run_pass(tim_hbm, tif_hbm, tim_tail, tif_tail,
             [(ibinc, ibinp, icnt, rowbuf, posb, out_i)])


_sc_gather = pl.kernel(
    _sc_body,
    out_type=[
        jax.ShapeDtypeStruct((_B, 128), jnp.float32),
        jax.ShapeDtypeStruct((_B, 128), jnp.float32),
        jax.ShapeDtypeStruct((_B, 128), jnp.float32),
    ],
    mesh=plsc.VectorSubcoreMesh(core_axis_name="c", subcore_axis_name="s",
                                num_cores=_NC, num_subcores=_NS),
    scratch_types=[
        pltpu.VMEM((_WIN,), jnp.int32),
        pltpu.VMEM((_WIN,), jnp.int32),
        pltpu.VMEM((_LCAP,), jnp.int32),
        pltpu.VMEM((_LCAP,), jnp.int32),
        pltpu.VMEM((_LCAP,), jnp.int32),
        pltpu.VMEM((_LCAP,), jnp.int32),
        pltpu.VMEM((_LCAP,), jnp.int32),
        pltpu.VMEM((_LCAP,), jnp.int32),
        pltpu.VMEM((_SPT, _CAP), jnp.int32),
        pltpu.VMEM((_SPT, _CAP), jnp.int32),
        pltpu.VMEM((_SPT, _CAP), jnp.int32),
        pltpu.VMEM((_SPT, _CAP), jnp.int32),
        pltpu.VMEM((_SPT, _CAP), jnp.int32),
        pltpu.VMEM((_SPT, _CAP), jnp.int32),
        pltpu.VMEM((64,), jnp.int32),
        pltpu.VMEM((64,), jnp.int32),
        pltpu.VMEM((64,), jnp.int32),
        pltpu.VMEM((_D, _SLABW), jnp.float32),
        pltpu.VMEM((_D, _SLABW), jnp.float32),
        pltpu.VMEM((_D, _SLABW), jnp.float32),
        pltpu.VMEM((_D, _SLABW), jnp.float32),
        pltpu.VMEM((_CAP, 128), jnp.float32),
        pltpu.VMEM((_CAP, 128), jnp.float32),
        pltpu.VMEM((_CAP,), jnp.int32),
        pltpu.VMEM((_CAP,), jnp.int32),
        pltpu.SemaphoreType.DMA,
        pltpu.SemaphoreType.DMA,
        pltpu.SemaphoreType.DMA,
    ],
    compiler_params=pltpu.CompilerParams(needs_layout_passes=False),
)

_R = 2048
_NSTEP = _B // _R


def _tc_body(gu, gc, gi, lab, w1u, w1i, b1, w2, b2, w3, b3, wamT, wamfT,
             ba, rating_out, scal_out, acc_ref):
    i = pl.program_id(0)

    @pl.when(i == 0)
    def _():
        acc_ref[0] = 0.0
        acc_ref[1] = 0.0

    f32 = jnp.float32
    u = gu[...]
    c = gc[...]
    um = u[:, :_D]
    uf = u[:, _D:2 * _D]
    im = gi[:, :_D]
    imf = gi[:, _D:2 * _D]
    mf = uf * imf
    h1 = jnp.maximum(
        jnp.dot(um, w1u[...], preferred_element_type=f32)
        + jnp.dot(im, w1i[...], preferred_element_type=f32) + b1[...], 0.0)
    h2 = jnp.maximum(jnp.dot(h1, w2[...], preferred_element_type=f32) + b2[...], 0.0)
    h3 = jnp.maximum(jnp.dot(h2, w3[...], preferred_element_type=f32) + b3[...], 0.0)
    dn = (((1,), (1,)), ((), ()))
    lrow = (lax.dot_general(wamT[...], h3, dn, preferred_element_type=f32)
            + lax.dot_general(wamfT[...], mf, dn, preferred_element_type=f32)
            + ba[0])
    y = lab[...]
    rating_out[...] = 1.0 / (1.0 + jnp.exp(-lrow))
    bce = jnp.maximum(lrow, 0.0) - lrow * y + jnp.log1p(jnp.exp(-jnp.abs(lrow)))
    d = u[:, :2 * _D] - c[:, :2 * _D]
    acc_ref[0] += jnp.sum(bce)
    acc_ref[1] += jnp.sum(d * d)

    @pl.when(i == _NSTEP - 1)
    def _():
        obce = acc_ref[0] / _B
        ot = acc_ref[1] / _B
        scal_out[0] = obce + ot
        scal_out[1] = obce
        scal_out[2] = ot


_tc_dense = pl.pallas_call(
    _tc_body,
    grid=(_NSTEP,),
    in_specs=[
        pl.BlockSpec((_R, 128), lambda i: (i, 0)),
        pl.BlockSpec((_R, 128), lambda i: (i, 0)),
        pl.BlockSpec((_R, 128), lambda i: (i, 0)),
        pl.BlockSpec((1, _R), lambda i: (0, i)),
        pl.BlockSpec((_D, _D), lambda i: (0, 0)),
        pl.BlockSpec((_D, _D), lambda i: (0, 0)),
        pl.BlockSpec((1, _D), lambda i: (0, 0)),
        pl.BlockSpec((_D, 16), lambda i: (0, 0)),
        pl.BlockSpec((1, 16), lambda i: (0, 0)),
        pl.BlockSpec((16, 8), lambda i: (0, 0)),
        pl.BlockSpec((1, 8), lambda i: (0, 0)),
        pl.BlockSpec((1, 8), lambda i: (0, 0)),
        pl.BlockSpec((1, _D), lambda i: (0, 0)),
        pl.BlockSpec(memory_space=pltpu.SMEM),
    ],
    out_specs=[
        pl.BlockSpec((1, _R), lambda i: (0, i)),
        pl.BlockSpec(memory_space=pltpu.SMEM),
    ],
    out_shape=[
        jax.ShapeDtypeStruct((1, _B), jnp.float32),
        jax.ShapeDtypeStruct((3,), jnp.float32),
    ],
    scratch_shapes=[pltpu.SMEM((2,), jnp.float32)],
)


def _tail(tT):
    w = _V - (_NSLAB - 1) * _SLABW
    return jnp.pad(tT[:, (_NSLAB - 1) * _SLABW:], ((0, 0), (0, _TAILW - w)))


def kernel(user_indices, item_indices, labels, emb_user_mlp, emb_item_mlp,
           emb_user_mf, emb_item_mf, W1, b1, W2, b2, W3, b3, Wa, ba, s2a_map):
    tum, tim, tuf, tif = (emb_user_mlp.T, emb_item_mlp.T,
                          emb_user_mf.T, emb_item_mf.T)
    gu, gc, gi = _sc_gather(
        user_indices, item_indices, tum, tim, tuf, tif, s2a_map,
        _tail(tum), _tail(tim), _tail(tuf), _tail(tif))
    rating_row, scal = _tc_dense(
        gu, gc, gi, labels.reshape(1, _B),
        W1[:_D], W1[_D:], b1.reshape(1, _D),
        W2, b2.reshape(1, 16), W3, b3.reshape(1, 8),
        Wa[:8].T, Wa[8:].T, ba)
    rating = rating_row.reshape(_B)
    return (scal[0], scal[1], scal[2], rating, labels)


# speedup vs baseline: 1.1250x; 1.0036x over previous
import jax
import jax.numpy as jnp
from jax import lax
from jax.experimental import pallas as pl
from jax.experimental.pallas import tpu as pltpu
from jax.experimental.pallas import tpu_sc as plsc

_B = 16384
_D = 32
_V = 1000000
_NC = 2
_NS = 16
_NW = _NC * _NS
_SLABW = 512
_SHIFT = 9
_NSLAB = (_V + _SLABW - 1) // _SLABW
_SPT = (_NSLAB + _NW - 1) // _NW
_WIN = 1024
_TAILW = 128
_LCAP = 768
_CAP = 32


def _sc_body(uidx_hbm, iidx_hbm, tum_hbm, tim_hbm, tuf_hbm, tif_hbm, s2a_hbm,
             tum_tail, tim_tail, tuf_tail, tif_tail,
             out_u, out_c, out_i,
             uwin, iwin,
             uvals, upos, cvals, cpos, ivals, ipos,
             ubinc, ubinp, cbinc, cbinp, ibinc, ibinp, ucnt, ccnt, icnt,
             sa0, sa1, sb0, sb1, rowbuf, rowbuf2, posb, posb2,
             semp0, semp1, sem2):
    t = lax.axis_index("s") * _NC + lax.axis_index("c")
    iota16 = lax.iota(jnp.int32, 16)
    neg1 = jnp.full((16,), -1, jnp.int32)

    def fire(tbl_a, tbl_b, tail_a, tail_b, sa, sb, semp, si):
        s = si * _NW + t

        @pl.when(s < _NSLAB - 1)
        def _():
            base = s * _SLABW
            for h in range(_SLABW // 128):
                pltpu.async_copy(tbl_a.at[:, pl.ds(base + h * 128, 128)],
                                 sa.at[:, pl.ds(h * 128, 128)], semp)
                pltpu.async_copy(tbl_b.at[:, pl.ds(base + h * 128, 128)],
                                 sb.at[:, pl.ds(h * 128, 128)], semp)

        @pl.when(s == _NSLAB - 1)
        def _():
            pltpu.async_copy(tail_a, sa.at[:, pl.ds(0, _TAILW)], semp)
            pltpu.async_copy(tail_b, sb.at[:, pl.ds(0, _TAILW)], semp)

    def drain(tbl_a, tail_a, sa, sb, semp, si):
        s = si * _NW + t

        @pl.when(s < _NSLAB - 1)
        def _():
            pltpu.make_async_copy(tbl_a.at[:, pl.ds(0, _SLABW)], sa, semp).wait()
            pltpu.make_async_copy(tbl_a.at[:, pl.ds(0, _SLABW)], sb, semp).wait()

        @pl.when(s == _NSLAB - 1)
        def _():
            pltpu.make_async_copy(tail_a, sa.at[:, pl.ds(0, _TAILW)], semp).wait()
            pltpu.make_async_copy(tail_a, sb.at[:, pl.ds(0, _TAILW)], semp).wait()


    fire(tum_hbm, tuf_hbm, tum_tail, tuf_tail, sa0, sb0, semp0, 0)
    fire(tum_hbm, tuf_hbm, tum_tail, tuf_tail, sa1, sb1, semp1, 1)

    def compact_window(base_b, win_ref, vals_ref, pos_ref, off):
        def chunk(k, off):
            v = win_ref[pl.ds(k * 16, 16)]
            s = lax.shift_right_logical(v, _SHIFT)
            m = (s & (_NW - 1)) == t
            cnt = plsc.all_reduce_population_count(m)
            wp = off + plsc.cumsum(m.astype(jnp.int32)) - 1
            plsc.store_scatter(vals_ref, [wp], v, mask=m)
            plsc.store_scatter(pos_ref, [wp], base_b + k * 16 + iota16, mask=m)
            return off + cnt[0]
        return lax.fori_loop(0, _WIN // 16, chunk, off, unroll=2)

    def awin(w, offs):
        uo, co, io = offs
        b0 = w * _WIN
        pltpu.sync_copy(uidx_hbm.at[pl.ds(b0, _WIN)], uwin)
        pltpu.sync_copy(iidx_hbm.at[pl.ds(b0, _WIN)], iwin)
        io = compact_window(b0, iwin, ivals, ipos, io)
        cps = [pltpu.async_copy(s2a_hbm.at[uwin.at[pl.ds(j * 128, 128)]],
                                iwin.at[pl.ds(j * 128, 128)], sem2)
               for j in range(_WIN // 128)]
        uo = compact_window(b0, uwin, uvals, upos, uo)
        for cp in cps:
            cp.wait()
        co = compact_window(b0, iwin, cvals, cpos, co)
        return (uo, co, io)

    nu, ncr, ni = lax.fori_loop(0, _B // _WIN, awin,
                                (jnp.int32(0), jnp.int32(0), jnp.int32(0)))

    zero16 = jnp.zeros((16,), jnp.int32)
    for cref in (ucnt, ccnt, icnt):
        for q in range(4):
            cref[pl.ds(q * 16, 16)] = zero16

    def bin_list(vals_ref, pos_ref, n, binc, binp, cnt_ref):
        def chunk(k, _):
            valid = k * 16 + iota16 < n
            v = vals_ref[pl.ds(k * 16, 16)]
            bp = pos_ref[pl.ds(k * 16, 16)]
            o = lax.shift_right_logical(v, 14)
            rank, lastm = plsc.scan_count(o, valid)
            base = plsc.load_gather(cnt_ref, [o], mask=valid)
            slot = jnp.minimum(base + rank - 1, _CAP - 1)
            plsc.store_scatter(binc, [o, slot], v & (_SLABW - 1), mask=valid)
            plsc.store_scatter(binp, [o, slot], bp, mask=valid)
            plsc.store_scatter(cnt_ref, [o], slot + 1, mask=lastm & valid)
            return 0

        lax.fori_loop(0, (n + 15) // 16, chunk, 0)

    bin_list(uvals, upos, nu, ubinc, ubinp, ucnt)
    bin_list(cvals, cpos, ncr, cbinc, cbinp, ccnt)
    bin_list(ivals, ipos, ni, ibinc, ibinp, icnt)

    def extract_set(si, sa, sb, binc, binp, cnt_ref, rb, pb, out_hbm):
        for q in range(_CAP // 16):
            pb[pl.ds(q * 16, 16)] = neg1
        cc = cnt_ref[pl.ds((si // 16) * 16, 16)]
        nsub = jnp.minimum(
            lax.reduce_max(jnp.where(iota16 == si % 16, cc, 0), (0,)), _CAP)

        def group(g, _):
            cols = binc[si, pl.ds(g * 16, 16)]
            bp = binp[si, pl.ds(g * 16, 16)]
            lanes = g * 16 + iota16
            gm = lanes < nsub
            plsc.store_scatter(pb, [lanes], bp, mask=gm)
            for d in range(_D):
                dd = jnp.broadcast_to(jnp.int32(d), (16,))
                va = plsc.load_gather(sa, [dd, cols], mask=gm)
                plsc.store_scatter(rb, [lanes, dd], va, mask=gm)
                vb = plsc.load_gather(sb, [dd, cols], mask=gm)
                plsc.store_scatter(rb, [lanes, dd + _D], vb, mask=gm)
            return 0

        lax.fori_loop(0, (nsub + 15) // 16, group, 0)
        return pltpu.async_copy(
            rb, out_hbm.at[plsc.Indices(pb, ignored_value=-1)], sem2)

    def run_pass(tbl_a, tbl_b, tail_a, tail_b, sets, primed_hi=-1):
        if primed_hi < 0:
            fire(tbl_a, tbl_b, tail_a, tail_b, sa0, sb0, semp0, 0)

        def unit(si, sa, sb, semp, nsa, nsb, nsemp):
            s = si * _NW + t

            @pl.when(((si + 1) * _NW + t < _NSLAB) & (si + 1 > primed_hi))
            def _():
                fire(tbl_a, tbl_b, tail_a, tail_b, nsa, nsb, nsemp, si + 1)

            @pl.when(s < _NSLAB)
            def _():
                drain(tbl_a, tail_a, sa, sb, semp, si)
                cps = [extract_set(si, sa, sb, *st) for st in sets]
                for cp in cps:
                    cp.wait()

        def outer(j, _):
            unit(2 * j, sa0, sb0, semp0, sa1, sb1, semp1)
            unit(2 * j + 1, sa1, sb1, semp1, sa0, sb0, semp0)
            return 0

        lax.fori_loop(0, _SPT // 2, outer, 0)

    run_pass(tum_hbm, tuf_hbm, tum_tail, tuf_tail,
             [(ubinc, ubinp, ucnt, rowbuf, posb, out_u),
              (cbinc, cbinp, ccnt, rowbuf2, posb2, out_c)], primed_hi=1)
    run_pass(tim_hbm, tif_hbm, tim_tail, tif_tail,
             [(ibinc, ibinp, icnt, rowbuf, posb, out_i)])


_sc_gather = pl.kernel(
    _sc_body,
    out_type=[
        jax.ShapeDtypeStruct((_B, 128), jnp.float32),
        jax.ShapeDtypeStruct((_B, 128), jnp.float32),
        jax.ShapeDtypeStruct((_B, 128), jnp.float32),
    ],
    mesh=plsc.VectorSubcoreMesh(core_axis_name="c", subcore_axis_name="s",
                                num_cores=_NC, num_subcores=_NS),
    scratch_types=[
        pltpu.VMEM((_WIN,), jnp.int32),
        pltpu.VMEM((_WIN,), jnp.int32),
        pltpu.VMEM((_LCAP,), jnp.int32),
        pltpu.VMEM((_LCAP,), jnp.int32),
        pltpu.VMEM((_LCAP,), jnp.int32),
        pltpu.VMEM((_LCAP,), jnp.int32),
        pltpu.VMEM((_LCAP,), jnp.int32),
        pltpu.VMEM((_LCAP,), jnp.int32),
        pltpu.VMEM((_SPT, _CAP), jnp.int32),
        pltpu.VMEM((_SPT, _CAP), jnp.int32),
        pltpu.VMEM((_SPT, _CAP), jnp.int32),
        pltpu.VMEM((_SPT, _CAP), jnp.int32),
        pltpu.VMEM((_SPT, _CAP), jnp.int32),
        pltpu.VMEM((_SPT, _CAP), jnp.int32),
        pltpu.VMEM((64,), jnp.int32),
        pltpu.VMEM((64,), jnp.int32),
        pltpu.VMEM((64,), jnp.int32),
        pltpu.VMEM((_D, _SLABW), jnp.float32),
        pltpu.VMEM((_D, _SLABW), jnp.float32),
        pltpu.VMEM((_D, _SLABW), jnp.float32),
        pltpu.VMEM((_D, _SLABW), jnp.float32),
        pltpu.VMEM((_CAP, 128), jnp.float32),
        pltpu.VMEM((_CAP, 128), jnp.float32),
        pltpu.VMEM((_CAP,), jnp.int32),
        pltpu.VMEM((_CAP,), jnp.int32),
        pltpu.SemaphoreType.DMA,
        pltpu.SemaphoreType.DMA,
        pltpu.SemaphoreType.DMA,
    ],
    compiler_params=pltpu.CompilerParams(needs_layout_passes=False),
)

_R = 2048
_NSTEP = _B // _R


def _tc_body(gu, gc, gi, lab, w1u, w1i, b1, w2, b2, w3, b3, wamT, wamfT,
             ba, rating_out, scal_out, acc_ref):
    i = pl.program_id(0)

    @pl.when(i == 0)
    def _():
        acc_ref[0] = 0.0
        acc_ref[1] = 0.0

    f32 = jnp.float32
    u = gu[...]
    c = gc[...]
    um = u[:, :_D]
    uf = u[:, _D:2 * _D]
    im = gi[:, :_D]
    imf = gi[:, _D:2 * _D]
    mf = uf * imf
    h1 = jnp.maximum(
        jnp.dot(um, w1u[...], preferred_element_type=f32)
        + jnp.dot(im, w1i[...], preferred_element_type=f32) + b1[...], 0.0)
    h2 = jnp.maximum(jnp.dot(h1, w2[...], preferred_element_type=f32) + b2[...], 0.0)
    h3 = jnp.maximum(jnp.dot(h2, w3[...], preferred_element_type=f32) + b3[...], 0.0)
    dn = (((1,), (1,)), ((), ()))
    lrow = (lax.dot_general(wamT[...], h3, dn, preferred_element_type=f32)
            + lax.dot_general(wamfT[...], mf, dn, preferred_element_type=f32)
            + ba[0])
    y = lab[...]
    rating_out[...] = 1.0 / (1.0 + jnp.exp(-lrow))
    bce = jnp.maximum(lrow, 0.0) - lrow * y + jnp.log1p(jnp.exp(-jnp.abs(lrow)))
    d = u[:, :2 * _D] - c[:, :2 * _D]
    acc_ref[0] += jnp.sum(bce)
    acc_ref[1] += jnp.sum(d * d)

    @pl.when(i == _NSTEP - 1)
    def _():
        obce = acc_ref[0] / _B
        ot = acc_ref[1] / _B
        scal_out[0] = obce + ot
        scal_out[1] = obce
        scal_out[2] = ot


_tc_dense = pl.pallas_call(
    _tc_body,
    grid=(_NSTEP,),
    in_specs=[
        pl.BlockSpec((_R, 128), lambda i: (i, 0)),
        pl.BlockSpec((_R, 128), lambda i: (i, 0)),
        pl.BlockSpec((_R, 128), lambda i: (i, 0)),
        pl.BlockSpec((1, _R), lambda i: (0, i)),
        pl.BlockSpec((_D, _D), lambda i: (0, 0)),
        pl.BlockSpec((_D, _D), lambda i: (0, 0)),
        pl.BlockSpec((1, _D), lambda i: (0, 0)),
        pl.BlockSpec((_D, 16), lambda i: (0, 0)),
        pl.BlockSpec((1, 16), lambda i: (0, 0)),
        pl.BlockSpec((16, 8), lambda i: (0, 0)),
        pl.BlockSpec((1, 8), lambda i: (0, 0)),
        pl.BlockSpec((1, 8), lambda i: (0, 0)),
        pl.BlockSpec((1, _D), lambda i: (0, 0)),
        pl.BlockSpec(memory_space=pltpu.SMEM),
    ],
    out_specs=[
        pl.BlockSpec((1, _R), lambda i: (0, i)),
        pl.BlockSpec(memory_space=pltpu.SMEM),
    ],
    out_shape=[
        jax.ShapeDtypeStruct((1, _B), jnp.float32),
        jax.ShapeDtypeStruct((3,), jnp.float32),
    ],
    scratch_shapes=[pltpu.SMEM((2,), jnp.float32)],
)


def _tail(tT):
    w = _V - (_NSLAB - 1) * _SLABW
    return jnp.pad(tT[:, (_NSLAB - 1) * _SLABW:], ((0, 0), (0, _TAILW - w)))


def kernel(user_indices, item_indices, labels, emb_user_mlp, emb_item_mlp,
           emb_user_mf, emb_item_mf, W1, b1, W2, b2, W3, b3, Wa, ba, s2a_map):
    tum, tim, tuf, tif = (emb_user_mlp.T, emb_item_mlp.T,
                          emb_user_mf.T, emb_item_mf.T)
    gu, gc, gi = _sc_gather(
        user_indices, item_indices, tum, tim, tuf, tif, s2a_map,
        _tail(tum), _tail(tim), _tail(tuf), _tail(tif))
    rating_row, scal = _tc_dense(
        gu, gc, gi, labels.reshape(1, _B),
        W1[:_D], W1[_D:], b1.reshape(1, _D),
        W2, b2.reshape(1, 16), W3, b3.reshape(1, 8),
        Wa[:8].T, Wa[8:].T, ba)
    rating = rating_row.reshape(_B)
    return (scal[0], scal[1], scal[2], rating, labels)
